# baseline (speedup 1.0000x reference)
import functools

import jax
import jax.numpy as jnp
from jax import lax
from jax.experimental import pallas as pl
from jax.experimental.pallas import tpu as pltpu
from jax.experimental.pallas import tpu_sc as plsc

NA = 50000
NB = 800000
NM = 2000
AF = 62
BF = 6
H = 20
OUT = 300
ND = 6
NL = 4

NAP = 53248
EP = 802816
MP = 2048
CH = 128
HP = 24
OP = 304
BLK = 512
NSUB = 16
NW = 32


def _mesh():
    return plsc.VectorSubcoreMesh(core_axis_name="c", subcore_axis_name="s",
                                  num_cores=2, num_subcores=NSUB)


def _sc_gather_scatter(table, src3, dst3, zrows, nbuf):
    W = table.shape[1]
    nch = src3.shape[1]
    ngr = nch // nbuf
    stripe = NAP // NSUB

    @functools.partial(
        pl.kernel,
        out_type=jax.ShapeDtypeStruct((2, NAP, W), jnp.float32),
        mesh=_mesh(),
        compiler_params=pltpu.CompilerParams(use_tc_tiling_on_sc=False),
        scratch_types=[
            pltpu.VMEM((nbuf, CH), jnp.int32),
            pltpu.VMEM((nbuf, CH), jnp.int32),
            pltpu.VMEM((nbuf, CH, W), jnp.float32),
            pltpu.VMEM_SHARED((NAP, W), jnp.float32),
            pltpu.SemaphoreType.DMA,
            pltpu.SemaphoreType.DMA,
        ],
    )
    def k(table_h, src_h, dst_h, z_h, out_h, ig, dg, rows, acc, semg, sems):
        c = lax.axis_index("c")
        s = lax.axis_index("s")
        w = c * NSUB + s
        pltpu.sync_copy(z_h, acc.at[pl.ds(s * stripe, stripe), :])
        plsc.subcore_barrier()

        def body(g, carry):
            pltpu.sync_copy(src_h.at[w, pl.ds(g * nbuf, nbuf), :], ig)
            pltpu.sync_copy(dst_h.at[w, pl.ds(g * nbuf, nbuf), :], dg)
            for b in range(nbuf):
                pltpu.async_copy(table_h.at[ig.at[b]], rows.at[b], semg)
            for b in range(nbuf):
                pltpu.make_async_copy(table_h.at[ig.at[b]], rows.at[b],
                                      semg).wait()
                pltpu.async_copy(rows.at[b], acc.at[dg.at[b]], sems,
                                 add=True)
            for b in range(nbuf):
                pltpu.make_async_copy(rows.at[b], acc.at[dg.at[b]],
                                      sems).wait()
            return carry

        lax.fori_loop(0, ngr, body, 0)
        plsc.subcore_barrier()
        pltpu.sync_copy(acc.at[pl.ds(s * stripe, stripe), :],
                        out_h.at[c, pl.ds(s * stripe, stripe), :])

    return k(table, src3, dst3, zrows)


def _sc_linear_scatter(rows_hbm, idx3, zrows, nacc, nbuf):
    W = rows_hbm.shape[1]
    nch = idx3.shape[1]
    ch = idx3.shape[2]
    ngr = nch // nbuf
    stripe = nacc // NSUB

    @functools.partial(
        pl.kernel,
        out_type=jax.ShapeDtypeStruct((2, nacc, W), jnp.float32),
        mesh=_mesh(),
        compiler_params=pltpu.CompilerParams(use_tc_tiling_on_sc=False),
        scratch_types=[
            pltpu.VMEM((nbuf, ch), jnp.int32),
            pltpu.VMEM((nbuf, ch, W), jnp.float32),
            pltpu.VMEM_SHARED((nacc, W), jnp.float32),
            pltpu.SemaphoreType.DMA,
            pltpu.SemaphoreType.DMA,
        ],
    )
    def k(rows_h, idx_h, z_h, out_h, dg, rows, acc, semg, sems):
        c = lax.axis_index("c")
        s = lax.axis_index("s")
        w = c * NSUB + s
        pltpu.sync_copy(z_h, acc.at[pl.ds(s * stripe, stripe), :])
        plsc.subcore_barrier()

        def body(g, carry):
            pltpu.sync_copy(idx_h.at[w, pl.ds(g * nbuf, nbuf), :], dg)
            for b in range(nbuf):
                off = (w * nch + g * nbuf + b) * ch
                pltpu.async_copy(rows_h.at[pl.ds(off, ch), :], rows.at[b],
                                 semg)
            for b in range(nbuf):
                off = (w * nch + g * nbuf + b) * ch
                pltpu.make_async_copy(rows_h.at[pl.ds(off, ch), :],
                                      rows.at[b], semg).wait()
                pltpu.async_copy(rows.at[b], acc.at[dg.at[b]], sems,
                                 add=True)
            for b in range(nbuf):
                pltpu.make_async_copy(rows.at[b], acc.at[dg.at[b]],
                                      sems).wait()
            return carry

        lax.fori_loop(0, ngr, body, 0)
        plsc.subcore_barrier()
        pltpu.sync_copy(acc.at[pl.ds(s * stripe, stripe), :],
                        out_h.at[c, pl.ds(s * stripe, stripe), :])

    return k(rows_hbm, idx3, zrows)


def _sc_fp_merge(p_list, idx3, zrows):
    nch = idx3.shape[1]
    ch = idx3.shape[2]
    nbuf = 2
    ngr = nch // nbuf
    stripe = MP // NSUB

    @functools.partial(
        pl.kernel,
        out_type=jax.ShapeDtypeStruct((2, MP, OP), jnp.float32),
        mesh=_mesh(),
        compiler_params=pltpu.CompilerParams(use_tc_tiling_on_sc=False),
        scratch_types=[
            pltpu.VMEM((nbuf, ch), jnp.int32),
            pltpu.VMEM((nbuf, ch, OP), jnp.float32),
            pltpu.VMEM_SHARED((MP, OP), jnp.float32),
            pltpu.SemaphoreType.DMA,
            pltpu.SemaphoreType.DMA,
        ],
    )
    def k(p0_h, p1_h, p2_h, p3_h, p4_h, idx_h, z_h, out_h,
          dg, rows, acc, semg, sems):
        c = lax.axis_index("c")
        s = lax.axis_index("s")
        w = c * NSUB + s
        pltpu.sync_copy(z_h, acc.at[pl.ds(s * stripe, stripe), :])
        plsc.subcore_barrier()

        for rows_h in (p0_h, p1_h, p2_h, p3_h, p4_h):
            def body(g, carry, rows_h=rows_h):
                pltpu.sync_copy(idx_h.at[w, pl.ds(g * nbuf, nbuf), :], dg)
                for b in range(nbuf):
                    off = (w * nch + g * nbuf + b) * ch
                    pltpu.async_copy(rows_h.at[pl.ds(off, ch), :],
                                     rows.at[b], semg)
                for b in range(nbuf):
                    off = (w * nch + g * nbuf + b) * ch
                    pltpu.make_async_copy(rows_h.at[pl.ds(off, ch), :],
                                          rows.at[b], semg).wait()
                    pltpu.async_copy(rows.at[b], acc.at[dg.at[b]], sems,
                                     add=True)
                for b in range(nbuf):
                    pltpu.make_async_copy(rows.at[b], acc.at[dg.at[b]],
                                          sems).wait()
                return carry

            lax.fori_loop(0, ngr, body, 0)

        plsc.subcore_barrier()
        pltpu.sync_copy(acc.at[pl.ds(s * stripe, stripe), :],
                        out_h.at[c, pl.ds(s * stripe, stripe), :])

    return k(*p_list, idx3, zrows)


def _softmax(z):
    z = z - jnp.max(z, axis=-1, keepdims=True)
    e = jnp.exp(z)
    return e / jnp.sum(e, axis=-1, keepdims=True)


def _full(shape):
    return pl.BlockSpec(shape, lambda i: (0,) * len(shape))


def _tl0(af, aggp, bondp, Ws, bs, WdA, WdB, Wfp0, bfp0, Wo, bo):
    def body(af_r, aggp_r, bondp_r, Ws_r, bs_r, WdA_r, WdB_r, Wfp0_r, bfp0_r,
             Wo_r, bo_r, acts_o, oh_o, pin_o, p0_o):
        x = af_r[...]
        pA, pB = aggp_r[0], aggp_r[1]
        agg = jnp.concatenate([pA[:, :31], pB[:, :31]], axis=1)
        cnt = pB[:, 31:32]
        deg = jnp.minimum(cnt, 5.0)
        lanes = lax.broadcasted_iota(jnp.int32, (BLK, 8), 1).astype(jnp.float32)
        oh = (lanes == deg).astype(jnp.float32)
        oh_o[...] = oh
        bag = bondp_r[0, :, :BF] + bondp_r[1, :, :BF]
        selfa = jnp.dot(x, Ws_r[...], preferred_element_type=jnp.float32)
        selfa = selfa + bs_r[...]
        neigh = jnp.zeros((BLK, H), jnp.float32)
        for d in range(ND):
            nd = jnp.dot(agg, WdA_r[d], preferred_element_type=jnp.float32)
            nd = nd + jnp.dot(bag, WdB_r[d], preferred_element_type=jnp.float32)
            neigh = neigh + oh[:, d:d + 1] * nd
        acts = jnp.maximum(selfa + neigh, 0.0)
        zc = jnp.zeros((BLK, HP - H), jnp.float32)
        zp = jnp.zeros((BLK, OP - OUT), jnp.float32)
        acts_o[...] = jnp.concatenate([acts, zc], axis=1)
        pin = _softmax(
            jnp.dot(x, Wfp0_r[...], preferred_element_type=jnp.float32)
            + bfp0_r[...])
        pin_o[...] = jnp.concatenate([pin, zp], axis=1)
        p0 = _softmax(
            jnp.dot(acts, Wo_r[...], preferred_element_type=jnp.float32)
            + bo_r[...])
        p0_o[...] = jnp.concatenate([p0, zp], axis=1)

    grid = (NAP // BLK,)
    return pl.pallas_call(
        body,
        grid=grid,
        in_specs=[
            pl.BlockSpec((BLK, AF), lambda i: (i, 0)),
            pl.BlockSpec((2, BLK, 32), lambda i: (0, i, 0)),
            pl.BlockSpec((2, BLK, 8), lambda i: (0, i, 0)),
            _full((AF, H)), _full((1, H)),
            _full((ND, AF, H)), _full((ND, BF, H)),
            _full((AF, OUT)), _full((1, OUT)),
            _full((H, OUT)), _full((1, OUT)),
        ],
        out_specs=[
            pl.BlockSpec((BLK, HP), lambda i: (i, 0)),
            pl.BlockSpec((BLK, 8), lambda i: (i, 0)),
            pl.BlockSpec((BLK, OP), lambda i: (i, 0)),
            pl.BlockSpec((BLK, OP), lambda i: (i, 0)),
        ],
        out_shape=[
            jax.ShapeDtypeStruct((NAP, HP), jnp.float32),
            jax.ShapeDtypeStruct((NAP, 8), jnp.float32),
            jax.ShapeDtypeStruct((NAP, OP), jnp.float32),
            jax.ShapeDtypeStruct((NAP, OP), jnp.float32),
        ],
    )(af, aggp, bondp, Ws, bs, WdA, WdB, Wfp0, bfp0, Wo, bo)


def _tl(acts_in, aggp, bondp, degoh, Ws, bs, WdA, WdB, Wo, bo):
    def body(x_r, aggp_r, bondp_r, oh_r, Ws_r, bs_r, WdA_r, WdB_r, Wo_r, bo_r,
             acts_o, p_o):
        x = x_r[...]
        agg = aggp_r[0] + aggp_r[1]
        bag = bondp_r[0, :, :BF] + bondp_r[1, :, :BF]
        oh = oh_r[...]
        selfa = jnp.dot(x, Ws_r[...], preferred_element_type=jnp.float32)
        selfa = selfa + bs_r[...]
        neigh = jnp.zeros((BLK, H), jnp.float32)
        for d in range(ND):
            nd = jnp.dot(agg, WdA_r[d], preferred_element_type=jnp.float32)
            nd = nd + jnp.dot(bag, WdB_r[d], preferred_element_type=jnp.float32)
            neigh = neigh + oh[:, d:d + 1] * nd
        acts = jnp.maximum(selfa + neigh, 0.0)
        zc = jnp.zeros((BLK, HP - H), jnp.float32)
        zp = jnp.zeros((BLK, OP - OUT), jnp.float32)
        acts_o[...] = jnp.concatenate([acts, zc], axis=1)
        p = _softmax(
            jnp.dot(acts, Wo_r[...], preferred_element_type=jnp.float32)
            + bo_r[...])
        p_o[...] = jnp.concatenate([p, zp], axis=1)

    grid = (NAP // BLK,)
    return pl.pallas_call(
        body,
        grid=grid,
        in_specs=[
            pl.BlockSpec((BLK, HP), lambda i: (i, 0)),
            pl.BlockSpec((2, BLK, HP), lambda i: (0, i, 0)),
            pl.BlockSpec((2, BLK, 8), lambda i: (0, i, 0)),
            pl.BlockSpec((BLK, 8), lambda i: (i, 0)),
            _full((HP, H)), _full((1, H)),
            _full((ND, HP, H)), _full((ND, BF, H)),
            _full((H, OUT)), _full((1, OUT)),
        ],
        out_specs=[
            pl.BlockSpec((BLK, HP), lambda i: (i, 0)),
            pl.BlockSpec((BLK, OP), lambda i: (i, 0)),
        ],
        out_shape=[
            jax.ShapeDtypeStruct((NAP, HP), jnp.float32),
            jax.ShapeDtypeStruct((NAP, OP), jnp.float32),
        ],
    )(acts_in, aggp, bondp, degoh, Ws, bs, WdA, WdB, Wo, bo)


def kernel(atom_features, bond_features, edge_index, molecule_ids,
           Wself0, bself0, Wself_rest, bself_rest,
           Wdeg0, Wdeg_rest, Wout0, bout0, Wout_layers, bout_layers):
    f32 = jnp.float32
    src = edge_index[0].astype(jnp.int32)
    dst = edge_index[1].astype(jnp.int32)

    srcp = jnp.concatenate([src, jnp.zeros((EP - NB,), jnp.int32)])
    dstp = jnp.concatenate([dst, jnp.full((EP - NB,), NA, jnp.int32)])
    molp = jnp.concatenate([molecule_ids.astype(jnp.int32),
                            jnp.full((NAP - NA,), NM, jnp.int32)])
    af_pad = jnp.zeros((NAP, AF), f32).at[:NA].set(atom_features)
    tableA = jnp.concatenate([af_pad[:, :31], jnp.zeros((NAP, 1), f32)], axis=1)
    tableB = jnp.concatenate([af_pad[:, 31:62], jnp.ones((NAP, 1), f32)], axis=1)
    table2 = jnp.concatenate([tableA, tableB], axis=0)
    bond_pad = jnp.zeros((EP, 8), f32).at[:NB, :BF].set(bond_features)

    z32 = jnp.zeros((NAP // NSUB, 32), f32)
    zH = jnp.zeros((NAP // NSUB, HP), f32)
    z8 = jnp.zeros((NAP // NSUB, 8), f32)
    zfp = jnp.zeros((MP // NSUB, OP), f32)

    src3 = srcp.reshape(NW, EP // NW // CH, CH)
    dst3 = dstp.reshape(NW, EP // NW // CH, CH)
    dst3_0 = dstp.reshape(NSUB, EP // NSUB // CH, CH)
    dst3_00 = jnp.concatenate([dst3_0, dst3_0], axis=0)
    src3_00 = jnp.concatenate(
        [srcp.reshape(NSUB, EP // NSUB // CH, CH),
         (srcp + NAP).reshape(NSUB, EP // NSUB // CH, CH)], axis=0)
    mol3 = molp.reshape(NW, 26, 64)

    bondp = _sc_linear_scatter(bond_pad, dst3, z8, NAP, 4)
    agg0p = _sc_gather_scatter(table2, src3_00, dst3_00, z32, 4)

    acts, degoh, probs_in, probs0 = _tl0(
        af_pad, agg0p, bondp,
        Wself0, bself0.reshape(1, H),
        Wdeg0[:, :AF, :], Wdeg0[:, AF:AF + BF, :],
        Wout0, bout0.reshape(1, OUT),
        Wout_layers[0], bout_layers[0].reshape(1, OUT))

    probs_list = [probs_in, probs0]

    zrow = jnp.zeros((1, H), f32)
    for l in range(1, NL):
        aggp = _sc_gather_scatter(acts, src3, dst3, zH, 4)
        Ws_p = jnp.concatenate(
            [Wself_rest[l - 1]] + [zrow] * (HP - H), axis=0)
        WdA_p = jnp.concatenate(
            [Wdeg_rest[l - 1][:, :H, :],
             jnp.zeros((ND, HP - H, H), f32)], axis=1)
        acts, probs = _tl(
            acts, aggp, bondp, degoh,
            Ws_p, bself_rest[l - 1].reshape(1, H),
            WdA_p, Wdeg_rest[l - 1][:, H:H + BF, :],
            Wout_layers[l], bout_layers[l].reshape(1, OUT))
        probs_list.append(probs)

    fpp = _sc_fp_merge(probs_list, mol3, zfp)
    return (fpp[0] + fpp[1])[:NM, :OUT]


# speedup vs baseline: 1.0992x; 1.0992x over previous
import functools

import jax
import jax.numpy as jnp
from jax import lax
from jax.experimental import pallas as pl
from jax.experimental.pallas import tpu as pltpu
from jax.experimental.pallas import tpu_sc as plsc

NA = 50000
NB = 800000
NM = 2000
AF = 62
BF = 6
H = 20
OUT = 300
ND = 6
NL = 4

NAP = 53248
EP = 802816
MP = 2048
CH = 128
HP = 24
OP = 304
BLK = 512
NSUB = 16
NW = 32


def _mesh():
    return plsc.VectorSubcoreMesh(core_axis_name="c", subcore_axis_name="s",
                                  num_cores=2, num_subcores=NSUB)


def _sc_gather_scatter(table, src3, dst3, zrows, nbuf):
    W = table.shape[1]
    nch = src3.shape[1]
    ngr = nch // nbuf
    stripe = NAP // NSUB

    @functools.partial(
        pl.kernel,
        out_type=jax.ShapeDtypeStruct((2, NAP, W), jnp.float32),
        mesh=_mesh(),
        compiler_params=pltpu.CompilerParams(use_tc_tiling_on_sc=False),
        scratch_types=[
            pltpu.VMEM((nbuf, CH), jnp.int32),
            pltpu.VMEM((nbuf, CH), jnp.int32),
            pltpu.VMEM((nbuf, CH, W), jnp.float32),
            pltpu.VMEM_SHARED((NAP, W), jnp.float32),
            pltpu.SemaphoreType.DMA,
            pltpu.SemaphoreType.DMA,
            pltpu.SemaphoreType.DMA,
        ],
    )
    def k(table_h, src_h, dst_h, z_h, out_h, ig, dg, rows, acc, semg, sems,
          semi):
        c = lax.axis_index("c")
        s = lax.axis_index("s")
        w = c * NSUB + s
        pltpu.sync_copy(z_h, acc.at[pl.ds(s * stripe, stripe), :])
        plsc.subcore_barrier()

        def body(g, carry):
            da = pltpu.async_copy(src_h.at[w, pl.ds(g * nbuf, nbuf), :],
                                  ig, semi)
            db = pltpu.async_copy(dst_h.at[w, pl.ds(g * nbuf, nbuf), :],
                                  dg, semi)
            da.wait()
            db.wait()
            for b in range(nbuf):
                pltpu.async_copy(table_h.at[ig.at[b]], rows.at[b], semg)
            for b in range(nbuf):
                pltpu.make_async_copy(table_h.at[ig.at[b]], rows.at[b],
                                      semg).wait()
                pltpu.async_copy(rows.at[b], acc.at[dg.at[b]], sems,
                                 add=True)
            for b in range(nbuf):
                pltpu.make_async_copy(rows.at[b], acc.at[dg.at[b]],
                                      sems).wait()
            return carry

        lax.fori_loop(0, ngr, body, 0)
        plsc.subcore_barrier()
        pltpu.sync_copy(acc.at[pl.ds(s * stripe, stripe), :],
                        out_h.at[c, pl.ds(s * stripe, stripe), :])

    return k(table, src3, dst3, zrows)


def _sc_linear_scatter(rows_hbm, idx3, zrows, nacc, nbuf):
    W = rows_hbm.shape[1]
    nch = idx3.shape[1]
    ch = idx3.shape[2]
    ngr = nch // nbuf
    stripe = nacc // NSUB

    @functools.partial(
        pl.kernel,
        out_type=jax.ShapeDtypeStruct((2, nacc, W), jnp.float32),
        mesh=_mesh(),
        compiler_params=pltpu.CompilerParams(use_tc_tiling_on_sc=False),
        scratch_types=[
            pltpu.VMEM((nbuf, ch), jnp.int32),
            pltpu.VMEM((nbuf, ch, W), jnp.float32),
            pltpu.VMEM_SHARED((nacc, W), jnp.float32),
            pltpu.SemaphoreType.DMA,
            pltpu.SemaphoreType.DMA,
        ],
    )
    def k(rows_h, idx_h, z_h, out_h, dg, rows, acc, semg, sems):
        c = lax.axis_index("c")
        s = lax.axis_index("s")
        w = c * NSUB + s
        pltpu.sync_copy(z_h, acc.at[pl.ds(s * stripe, stripe), :])
        plsc.subcore_barrier()

        def body(g, carry):
            pltpu.sync_copy(idx_h.at[w, pl.ds(g * nbuf, nbuf), :], dg)
            for b in range(nbuf):
                off = (w * nch + g * nbuf + b) * ch
                pltpu.async_copy(rows_h.at[pl.ds(off, ch), :], rows.at[b],
                                 semg)
            for b in range(nbuf):
                off = (w * nch + g * nbuf + b) * ch
                pltpu.make_async_copy(rows_h.at[pl.ds(off, ch), :],
                                      rows.at[b], semg).wait()
                pltpu.async_copy(rows.at[b], acc.at[dg.at[b]], sems,
                                 add=True)
            for b in range(nbuf):
                pltpu.make_async_copy(rows.at[b], acc.at[dg.at[b]],
                                      sems).wait()
            return carry

        lax.fori_loop(0, ngr, body, 0)
        plsc.subcore_barrier()
        pltpu.sync_copy(acc.at[pl.ds(s * stripe, stripe), :],
                        out_h.at[c, pl.ds(s * stripe, stripe), :])

    return k(rows_hbm, idx3, zrows)


def _softmax(z):
    z = z - jnp.max(z, axis=-1, keepdims=True)
    e = jnp.exp(z)
    return e / jnp.sum(e, axis=-1, keepdims=True)


def _full(shape):
    return pl.BlockSpec(shape, lambda i: (0,) * len(shape))


def _tl0(af, aggp, bondp, Ws, bs, WdA, WdB, Wfp0, bfp0, Wo, bo):
    def body(af_r, aggp_r, bondp_r, Ws_r, bs_r, WdA_r, WdB_r, Wfp0_r, bfp0_r,
             Wo_r, bo_r, acts_o, oh_o, pin_o, p0_o):
        x = af_r[...]
        pA, pB = aggp_r[0], aggp_r[1]
        agg = jnp.concatenate([pA[:, :31], pB[:, :31]], axis=1)
        cnt = pB[:, 31:32]
        deg = jnp.minimum(cnt, 5.0)
        lanes = lax.broadcasted_iota(jnp.int32, (BLK, 8), 1).astype(jnp.float32)
        oh = (lanes == deg).astype(jnp.float32)
        oh_o[...] = oh
        bag = bondp_r[0, :, :BF] + bondp_r[1, :, :BF]
        selfa = jnp.dot(x, Ws_r[...], preferred_element_type=jnp.float32)
        selfa = selfa + bs_r[...]
        neigh = jnp.zeros((BLK, H), jnp.float32)
        for d in range(ND):
            nd = jnp.dot(agg, WdA_r[d], preferred_element_type=jnp.float32)
            nd = nd + jnp.dot(bag, WdB_r[d], preferred_element_type=jnp.float32)
            neigh = neigh + oh[:, d:d + 1] * nd
        acts = jnp.maximum(selfa + neigh, 0.0)
        zc = jnp.zeros((BLK, HP - H), jnp.float32)
        zp = jnp.zeros((BLK, OP - OUT), jnp.float32)
        acts_o[...] = jnp.concatenate([acts, zc], axis=1)
        pin = _softmax(
            jnp.dot(x, Wfp0_r[...], preferred_element_type=jnp.float32)
            + bfp0_r[...])
        pin_o[...] = jnp.concatenate([pin, zp], axis=1)
        p0 = _softmax(
            jnp.dot(acts, Wo_r[...], preferred_element_type=jnp.float32)
            + bo_r[...])
        p0_o[...] = jnp.concatenate([p0, zp], axis=1)

    grid = (NAP // BLK,)
    return pl.pallas_call(
        body,
        grid=grid,
        in_specs=[
            pl.BlockSpec((BLK, AF), lambda i: (i, 0)),
            pl.BlockSpec((2, BLK, 32), lambda i: (0, i, 0)),
            pl.BlockSpec((2, BLK, 8), lambda i: (0, i, 0)),
            _full((AF, H)), _full((1, H)),
            _full((ND, AF, H)), _full((ND, BF, H)),
            _full((AF, OUT)), _full((1, OUT)),
            _full((H, OUT)), _full((1, OUT)),
        ],
        out_specs=[
            pl.BlockSpec((BLK, HP), lambda i: (i, 0)),
            pl.BlockSpec((BLK, 8), lambda i: (i, 0)),
            pl.BlockSpec((BLK, OP), lambda i: (i, 0)),
            pl.BlockSpec((BLK, OP), lambda i: (i, 0)),
        ],
        out_shape=[
            jax.ShapeDtypeStruct((NAP, HP), jnp.float32),
            jax.ShapeDtypeStruct((NAP, 8), jnp.float32),
            jax.ShapeDtypeStruct((NAP, OP), jnp.float32),
            jax.ShapeDtypeStruct((NAP, OP), jnp.float32),
        ],
    )(af, aggp, bondp, Ws, bs, WdA, WdB, Wfp0, bfp0, Wo, bo)


def _tl(acts_in, aggp, bondp, degoh, Ws, bs, WdA, WdB, Wo, bo):
    def body(x_r, aggp_r, bondp_r, oh_r, Ws_r, bs_r, WdA_r, WdB_r, Wo_r, bo_r,
             acts_o, p_o):
        x = x_r[...]
        agg = aggp_r[0] + aggp_r[1]
        bag = bondp_r[0, :, :BF] + bondp_r[1, :, :BF]
        oh = oh_r[...]
        selfa = jnp.dot(x, Ws_r[...], preferred_element_type=jnp.float32)
        selfa = selfa + bs_r[...]
        neigh = jnp.zeros((BLK, H), jnp.float32)
        for d in range(ND):
            nd = jnp.dot(agg, WdA_r[d], preferred_element_type=jnp.float32)
            nd = nd + jnp.dot(bag, WdB_r[d], preferred_element_type=jnp.float32)
            neigh = neigh + oh[:, d:d + 1] * nd
        acts = jnp.maximum(selfa + neigh, 0.0)
        zc = jnp.zeros((BLK, HP - H), jnp.float32)
        zp = jnp.zeros((BLK, OP - OUT), jnp.float32)
        acts_o[...] = jnp.concatenate([acts, zc], axis=1)
        p = _softmax(
            jnp.dot(acts, Wo_r[...], preferred_element_type=jnp.float32)
            + bo_r[...])
        p_o[...] = jnp.concatenate([p, zp], axis=1)

    grid = (NAP // BLK,)
    return pl.pallas_call(
        body,
        grid=grid,
        in_specs=[
            pl.BlockSpec((BLK, HP), lambda i: (i, 0)),
            pl.BlockSpec((2, BLK, HP), lambda i: (0, i, 0)),
            pl.BlockSpec((2, BLK, 8), lambda i: (0, i, 0)),
            pl.BlockSpec((BLK, 8), lambda i: (i, 0)),
            _full((HP, H)), _full((1, H)),
            _full((ND, HP, H)), _full((ND, BF, H)),
            _full((H, OUT)), _full((1, OUT)),
        ],
        out_specs=[
            pl.BlockSpec((BLK, HP), lambda i: (i, 0)),
            pl.BlockSpec((BLK, OP), lambda i: (i, 0)),
        ],
        out_shape=[
            jax.ShapeDtypeStruct((NAP, HP), jnp.float32),
            jax.ShapeDtypeStruct((NAP, OP), jnp.float32),
        ],
    )(acts_in, aggp, bondp, degoh, Ws, bs, WdA, WdB, Wo, bo)


def kernel(atom_features, bond_features, edge_index, molecule_ids,
           Wself0, bself0, Wself_rest, bself_rest,
           Wdeg0, Wdeg_rest, Wout0, bout0, Wout_layers, bout_layers):
    f32 = jnp.float32
    src = edge_index[0].astype(jnp.int32)
    dst = edge_index[1].astype(jnp.int32)

    srcp = jnp.concatenate([src, jnp.zeros((EP - NB,), jnp.int32)])
    dstp = jnp.concatenate([dst, jnp.full((EP - NB,), NA, jnp.int32)])
    molp = jnp.concatenate([molecule_ids.astype(jnp.int32),
                            jnp.full((NAP - NA,), NM, jnp.int32)])
    af_pad = jnp.zeros((NAP, AF), f32).at[:NA].set(atom_features)
    tableA = jnp.concatenate([af_pad[:, :31], jnp.zeros((NAP, 1), f32)], axis=1)
    tableB = jnp.concatenate([af_pad[:, 31:62], jnp.ones((NAP, 1), f32)], axis=1)
    table2 = jnp.concatenate([tableA, tableB], axis=0)
    bond_pad = jnp.zeros((EP, 8), f32).at[:NB, :BF].set(bond_features)

    z32 = jnp.zeros((NAP // NSUB, 32), f32)
    zH = jnp.zeros((NAP // NSUB, HP), f32)
    z8 = jnp.zeros((NAP // NSUB, 8), f32)
    zfp = jnp.zeros((MP // NSUB, OP), f32)

    src3 = srcp.reshape(NW, EP // NW // CH, CH)
    dst3 = dstp.reshape(NW, EP // NW // CH, CH)
    dst3_0 = dstp.reshape(NSUB, EP // NSUB // CH, CH)
    dst3_00 = jnp.concatenate([dst3_0, dst3_0], axis=0)
    src3_00 = jnp.concatenate(
        [srcp.reshape(NSUB, EP // NSUB // CH, CH),
         (srcp + NAP).reshape(NSUB, EP // NSUB // CH, CH)], axis=0)
    mol3 = molp.reshape(NW, 26, 64)

    bondp = _sc_linear_scatter(bond_pad, dst3, z8, NAP, 7)
    agg0p = _sc_gather_scatter(table2, src3_00, dst3_00, z32, 4)

    acts, degoh, probs_in, probs0 = _tl0(
        af_pad, agg0p, bondp,
        Wself0, bself0.reshape(1, H),
        Wdeg0[:, :AF, :], Wdeg0[:, AF:AF + BF, :],
        Wout0, bout0.reshape(1, OUT),
        Wout_layers[0], bout_layers[0].reshape(1, OUT))

    fp_parts = [_sc_linear_scatter(probs_in, mol3, zfp, MP, 2),
                _sc_linear_scatter(probs0, mol3, zfp, MP, 2)]

    zrow = jnp.zeros((1, H), f32)
    for l in range(1, NL):
        aggp = _sc_gather_scatter(acts, src3, dst3, zH, 7)
        Ws_p = jnp.concatenate(
            [Wself_rest[l - 1]] + [zrow] * (HP - H), axis=0)
        WdA_p = jnp.concatenate(
            [Wdeg_rest[l - 1][:, :H, :],
             jnp.zeros((ND, HP - H, H), f32)], axis=1)
        acts, probs = _tl(
            acts, aggp, bondp, degoh,
            Ws_p, bself_rest[l - 1].reshape(1, H),
            WdA_p, Wdeg_rest[l - 1][:, H:H + BF, :],
            Wout_layers[l], bout_layers[l].reshape(1, OUT))
        fp_parts.append(_sc_linear_scatter(probs, mol3, zfp, MP, 2))

    fp = sum(p[0] + p[1] for p in fp_parts)
    return fp[:NM, :OUT]
